# baseline (speedup 1.0000x reference)
import jax
import jax.numpy as jnp
from jax.experimental import pallas as pl
from jax.experimental.pallas import tpu as pltpu

_BK = 8192


def _copy_kernel(in_ref, out_ref):
    out_ref[...] = in_ref[...]


def kernel(feature, item_embedding, W_ih, W_hh, b_ih, b_hh, h0):
    M, D = item_embedding.shape
    t = item_embedding.reshape(M // 2, D * 2)
    grid = ((M // 2) // _BK,)
    out = pl.pallas_call(
        _copy_kernel,
        out_shape=jax.ShapeDtypeStruct((M // 2, D * 2), item_embedding.dtype),
        grid=grid,
        in_specs=[pl.BlockSpec((_BK, D * 2), lambda g: (g, 0))],
        out_specs=pl.BlockSpec((_BK, D * 2), lambda g: (g, 0)),
    )(t)
    return out.reshape(M, D)


# speedup vs baseline: 1.2618x; 1.2618x over previous
import functools
import jax
import jax.numpy as jnp
from jax import lax
from jax.experimental import pallas as pl
from jax.experimental.pallas import tpu as pltpu
from jax.experimental.pallas import tpu_sc as plsc

_NC = 2
_NS = 16
_NW = _NC * _NS
_CH = 488
_NCHUNK_PER_TILE = 64


def _sc_copy(tab_in, tab_out, buf, in_sems, out_sems):
    M = tab_in.shape[0]
    wid = lax.axis_index("s") * _NC + lax.axis_index("c")
    tail_start = (M - _CH) // 8 * 8

    def chunk_start(j):
        k = wid + j * _NW
        s = jnp.where(j < _NCHUNK_PER_TILE, k * _CH, tail_start)
        return pl.multiple_of(s, 8)

    nch = _NCHUNK_PER_TILE + 1
    for j in range(nch):
        s = chunk_start(j)
        pltpu.sync_copy(tab_in.at[pl.ds(s, _CH), :], buf.at[0])
        pltpu.sync_copy(buf.at[0], tab_out.at[pl.ds(s, _CH), :])


def kernel(feature, item_embedding, W_ih, W_hh, b_ih, b_hh, h0):
    M, D = item_embedding.shape
    mesh = plsc.VectorSubcoreMesh(core_axis_name="c", subcore_axis_name="s")
    f = functools.partial(
        pl.kernel,
        mesh=mesh,
        out_type=jax.ShapeDtypeStruct((M, D), item_embedding.dtype),
        scratch_types=[
            pltpu.VMEM((2, _CH, D), jnp.float32),
            pltpu.SemaphoreType.DMA((2,)),
            pltpu.SemaphoreType.DMA((2,)),
        ],
    )(_sc_copy)
    return f(item_embedding)


# speedup vs baseline: 1.8593x; 1.4736x over previous
import jax
import jax.numpy as jnp
from jax.experimental import pallas as pl
from jax.experimental.pallas import tpu as pltpu


def _gather_rnn_kernel(feature_smem, idxs_vmem, w_cat, bias, h0_ref,
                       table_in, upd_out, sem):
    L = idxs_vmem.shape[0]

    def gather_start(i, _):
        idx = feature_smem[i, 0]
        pltpu.make_async_copy(table_in.at[pl.ds(idx, 1), :],
                              upd_out.at[pl.ds(i, 1), :], sem).start()
        return 0

    jax.lax.fori_loop(0, L, gather_start, 0)

    def gather_wait(i, _):
        idx = feature_smem[i, 0]
        pltpu.make_async_copy(table_in.at[pl.ds(idx, 1), :],
                              upd_out.at[pl.ds(i, 1), :], sem).wait()
        return 0

    jax.lax.fori_loop(0, L, gather_wait, 0)

    wc = w_cat[...]
    b = bias[...]
    idxs = idxs_vmem[...]

    def step(i, h):
        x = upd_out[pl.ds(i, 1), :]
        xh = jnp.concatenate([x, h], axis=1)
        h_new = jnp.tanh(
            jnp.dot(xh, wc, preferred_element_type=jnp.float32) + b)
        t_i = feature_smem[i, 1]
        prev = jnp.where(i == 0, L - 1, i - 1)
        dt = (t_i - feature_smem[prev, 1]).astype(jnp.float32)
        idx_i = feature_smem[i, 0]
        upd_out[...] = jnp.where(idxs == idx_i, h_new, upd_out[...])
        return h_new * (1.0 / dt + 1.0)

    jax.lax.fori_loop(0, L, step, h0_ref[...])


def _scatter_kernel(feature_smem, upd_vmem, table_in, table_out, sem):
    L = feature_smem.shape[0]

    def scatter_start(i, _):
        idx = feature_smem[i, 0]
        pltpu.make_async_copy(upd_vmem.at[pl.ds(i, 1), :],
                              table_out.at[pl.ds(idx, 1), :], sem).start()
        return 0

    jax.lax.fori_loop(0, L, scatter_start, 0)

    def scatter_wait(i, _):
        idx = feature_smem[i, 0]
        pltpu.make_async_copy(upd_vmem.at[pl.ds(i, 1), :],
                              table_out.at[pl.ds(idx, 1), :], sem).wait()
        return 0

    jax.lax.fori_loop(0, L, scatter_wait, 0)


def kernel(feature, item_embedding, W_ih, W_hh, b_ih, b_hh, h0):
    L = feature.shape[0]
    M, D = item_embedding.shape
    H = W_ih.shape[0]
    w_cat = jnp.concatenate([W_ih, W_hh], axis=1).T
    bias = (b_ih + b_hh).reshape(1, H)
    idxs2d = feature[:, 0:1]
    h02d = h0.reshape(1, H)

    updates = pl.pallas_call(
        _gather_rnn_kernel,
        out_shape=jax.ShapeDtypeStruct((L, D), jnp.float32),
        in_specs=[
            pl.BlockSpec(memory_space=pltpu.MemorySpace.SMEM),
            pl.BlockSpec(memory_space=pltpu.MemorySpace.VMEM),
            pl.BlockSpec(memory_space=pltpu.MemorySpace.VMEM),
            pl.BlockSpec(memory_space=pltpu.MemorySpace.VMEM),
            pl.BlockSpec(memory_space=pltpu.MemorySpace.VMEM),
            pl.BlockSpec(memory_space=pltpu.MemorySpace.HBM),
        ],
        out_specs=pl.BlockSpec(memory_space=pltpu.MemorySpace.VMEM),
        scratch_shapes=[pltpu.SemaphoreType.DMA],
    )(feature, idxs2d, w_cat, bias, h02d, item_embedding)

    return pl.pallas_call(
        _scatter_kernel,
        out_shape=jax.ShapeDtypeStruct((M, D), item_embedding.dtype),
        in_specs=[
            pl.BlockSpec(memory_space=pltpu.MemorySpace.SMEM),
            pl.BlockSpec(memory_space=pltpu.MemorySpace.VMEM),
            pl.BlockSpec(memory_space=pltpu.MemorySpace.HBM),
        ],
        out_specs=pl.BlockSpec(memory_space=pltpu.MemorySpace.HBM),
        input_output_aliases={2: 0},
        scratch_shapes=[pltpu.SemaphoreType.DMA],
    )(feature, updates, item_embedding)


# speedup vs baseline: 1.9078x; 1.0261x over previous
import jax
import jax.numpy as jnp
from jax.experimental import pallas as pl
from jax.experimental.pallas import tpu as pltpu


def _rnn_update_kernel(feature_smem, idxs_vmem, wih_t, whh_t, bias, h0_ref,
                       table_in, table_out, buf, a_scr, sem):
    L = idxs_vmem.shape[0]

    for i in range(L):
        idx = feature_smem[i, 0]
        pltpu.make_async_copy(table_out.at[pl.ds(idx, 1), :],
                              buf.at[pl.ds(i, 1), :], sem).start()
    for i in range(L):
        idx = feature_smem[i, 0]
        pltpu.make_async_copy(table_out.at[pl.ds(idx, 1), :],
                              buf.at[pl.ds(i, 1), :], sem).wait()

    wih = wih_t[...]
    whh = whh_t[...]
    b = bias[...]
    idxs = idxs_vmem[...]
    a_scr[...] = jnp.dot(buf[...], wih, preferred_element_type=jnp.float32) + b

    h = h0_ref[...]
    for i in range(L):
        pre = a_scr[i:i + 1, :] + jnp.dot(h, whh,
                                          preferred_element_type=jnp.float32)
        h_new = jnp.tanh(pre)
        dt = (feature_smem[i, 1]
              - feature_smem[(i - 1) % L, 1]).astype(jnp.float32)
        h = h_new * (1.0 / dt + 1.0)
        mask = idxs == feature_smem[i, 0]
        buf[...] = jnp.where(mask, h_new, buf[...])
        a_new = jnp.dot(h_new, wih, preferred_element_type=jnp.float32) + b
        a_scr[...] = jnp.where(mask, a_new, a_scr[...])

    for i in range(L):
        idx = feature_smem[i, 0]
        pltpu.make_async_copy(buf.at[pl.ds(i, 1), :],
                              table_out.at[pl.ds(idx, 1), :], sem).start()
    for i in range(L):
        idx = feature_smem[i, 0]
        pltpu.make_async_copy(buf.at[pl.ds(i, 1), :],
                              table_out.at[pl.ds(idx, 1), :], sem).wait()


def kernel(feature, item_embedding, W_ih, W_hh, b_ih, b_hh, h0):
    L = feature.shape[0]
    M, D = item_embedding.shape
    H = W_ih.shape[0]
    wih_t = W_ih.T
    whh_t = W_hh.T
    bias = (b_ih + b_hh).reshape(1, H)
    idxs2d = feature[:, 0:1]
    h02d = h0.reshape(1, H)

    return pl.pallas_call(
        _rnn_update_kernel,
        out_shape=jax.ShapeDtypeStruct((M, D), item_embedding.dtype),
        in_specs=[
            pl.BlockSpec(memory_space=pltpu.MemorySpace.SMEM),
            pl.BlockSpec(memory_space=pltpu.MemorySpace.VMEM),
            pl.BlockSpec(memory_space=pltpu.MemorySpace.VMEM),
            pl.BlockSpec(memory_space=pltpu.MemorySpace.VMEM),
            pl.BlockSpec(memory_space=pltpu.MemorySpace.VMEM),
            pl.BlockSpec(memory_space=pltpu.MemorySpace.VMEM),
            pl.BlockSpec(memory_space=pltpu.MemorySpace.HBM),
        ],
        out_specs=pl.BlockSpec(memory_space=pltpu.MemorySpace.HBM),
        input_output_aliases={6: 0},
        scratch_shapes=[
            pltpu.VMEM((L, D), jnp.float32),
            pltpu.VMEM((L, D), jnp.float32),
            pltpu.SemaphoreType.DMA,
        ],
    )(feature, idxs2d, wih_t, whh_t, bias, h02d, item_embedding)
